# Initial kernel scaffold; baseline (speedup 1.0000x reference)
import jax
import jax.numpy as jnp
from jax.experimental import pallas as pl


def kernel(conv1_w, conv1_b, conv2_w, conv2_b, conv3_w, conv3_b, conv4_w, conv4_b, conv5_w, conv5_b, linear0_w, linear0_b, linear1_w, linear1_b, x):
    raise NotImplementedError("write your pallas kernel here")


# speedup vs baseline: 4.2617x; 4.2617x over previous
import numpy as np

import jax
import jax.numpy as jnp
from jax.experimental import pallas as pl
from jax.experimental.pallas import tpu as pltpu

_VMEM_LIMIT = 60 * 1024 * 1024


def _elu(x):
    return jnp.where(x > 0.0, x, jnp.exp(jnp.minimum(x, 0.0)) - 1.0)


def _sel_np(n_in, n_out, p):
    r = np.arange(n_in)[:, None]
    c = np.arange(n_out)[None, :]
    return (r == p * c).astype(np.float32)


_SW1 = _sel_np(123, 128, 2)
_SH1 = _sel_np(184, 96, 2)
_SW2 = _sel_np(128, 24, 3)
_SH2 = _sel_np(88, 32, 3)
_S3F = np.zeros((592, 72), np.float32)
for _ho in range(9):
    for _wo in range(6):
        _S3F[(3 * _ho) * 24 + 3 * _wo, _ho * 8 + _wo] = 1.0
_r = np.arange(52)[:, None]
_q = np.arange(3584)[None, :] % 28
_BMAT = (_r == (_q // 4) * 8 + (_q % 4)).astype(np.float32)
_TMASK = (np.arange(128)[:, None] == np.arange(3584)[None, :] // 28
          ).astype(np.float32)
_ONES8 = np.ones((8, 128), np.float32)


def _fwd_kernel(x_ref, w1_ref, b1_ref, w2_ref, b2_ref, w3_ref, b3_ref,
                w4_ref, b4_ref, w5_ref, b5_ref, l0w_ref, l0b_ref,
                l1w_ref, l1b_ref, sw1_ref, sh1_ref, sw2_ref, sh2_ref,
                s3f_ref, bmat_ref, tmask_ref, ones8_ref, o_ref):
    f32 = jnp.float32
    x = 100.0 * (1.0 - x_ref[0, 0])

    w1 = w1_ref[...]
    xs = [x[:, kw:kw + 124] for kw in range(5)]
    acc = jnp.zeros((8, 178, 124), f32) + b1_ref[...][:, :, None]
    for kh in range(5):
        for kw in range(5):
            t = kh * 5 + kw
            acc = acc + w1[:, t:t + 1][:, :, None] * xs[kw][kh:kh + 178][None]

    m = jnp.maximum(jnp.maximum(acc[:, 0:177, 0:123], acc[:, 0:177, 1:124]),
                    jnp.maximum(acc[:, 1:178, 0:123], acc[:, 1:178, 1:124]))
    m = jnp.concatenate([m, jnp.zeros((8, 7, 123), f32)], axis=1)
    z = jnp.dot(m.reshape(1472, 123), sw1_ref[...],
                preferred_element_type=f32)
    z = jnp.transpose(z.reshape(8, 184, 128), (0, 2, 1))
    z = jnp.dot(z.reshape(1024, 184), sh1_ref[...],
                preferred_element_type=f32)
    z = jnp.transpose(z.reshape(8, 128, 96), (0, 2, 1))
    y = _elu(z).reshape(8, 96 * 128)

    l2 = 86 * 128 + 60
    pat = jnp.concatenate(
        [y[:, kh * 128 + kw:kh * 128 + kw + l2]
         for kh in range(3) for kw in range(3)], axis=0)
    y = jnp.dot(w2_ref[...], pat, preferred_element_type=f32) + b2_ref[...]
    y = jnp.concatenate([y, jnp.zeros((64, 88 * 128 - l2), f32)], axis=1)

    lm2 = 84 * 128 + 58
    m = None
    for i in range(3):
        for j in range(3):
            s = y[:, i * 128 + j:i * 128 + j + lm2]
            m = s if m is None else jnp.maximum(m, s)
    m = jnp.concatenate([m, jnp.zeros((64, 88 * 128 - lm2), f32)], axis=1)
    z = jnp.dot(m.reshape(5632, 128), sw2_ref[...],
                preferred_element_type=f32)
    z = jnp.transpose(z.reshape(64, 88, 24), (0, 2, 1))
    z = jnp.dot(z.reshape(1536, 88), sh2_ref[...],
                preferred_element_type=f32)
    z = jnp.transpose(z.reshape(64, 24, 32), (0, 2, 1))
    y = _elu(z).reshape(64, 32 * 24)

    l3 = 26 * 24 + 18
    acc3 = None
    for kh in range(3):
        for kw in range(3):
            t = kh * 3 + kw
            off = kh * 24 + kw
            c = jnp.dot(w3_ref[t], y[:, off:off + l3],
                        preferred_element_type=f32)
            acc3 = c if acc3 is None else acc3 + c
    y = acc3 + b3_ref[...]

    lm3 = 24 * 24 + 16
    m = None
    for i in range(3):
        for j in range(3):
            s = y[:, i * 24 + j:i * 24 + j + lm3]
            m = s if m is None else jnp.maximum(m, s)
    y = _elu(jnp.dot(m, s3f_ref[...], preferred_element_type=f32))

    l4 = 6 * 8 + 4
    acc4 = None
    for kh in range(3):
        for kw in range(3):
            t = kh * 3 + kw
            off = kh * 8 + kw
            c = jnp.dot(w4_ref[t], y[:, off:off + l4],
                        preferred_element_type=f32)
            acc4 = c if acc4 is None else acc4 + c
    y = _elu(acc4 + b4_ref[...])

    h = jnp.dot(w5_ref[...], y, preferred_element_type=f32) + b5_ref[...]

    hh = jnp.dot(h, bmat_ref[...], preferred_element_type=f32)
    hh = hh * tmask_ref[...]
    flat8 = jnp.dot(ones8_ref[...], hh,
                    preferred_element_type=f32)

    h0 = jnp.dot(flat8.astype(jnp.bfloat16), l0w_ref[...],
                 preferred_element_type=f32)
    h0 = _elu(h0 + l0b_ref[...])
    out = jnp.dot(h0.astype(jnp.bfloat16), l1w_ref[...],
                  preferred_element_type=f32)
    o_ref[...] = (out + l1b_ref[...])[0:1]


def kernel(conv1_w, conv1_b, conv2_w, conv2_b, conv3_w, conv3_b,
           conv4_w, conv4_b, conv5_w, conv5_b,
           linear0_w, linear0_b, linear1_w, linear1_b, x):
    w2k = jnp.transpose(conv2_w.reshape(64, 8, 9), (0, 2, 1)).reshape(64, 72)
    w3t = jnp.transpose(conv3_w.reshape(64, 64, 9), (2, 0, 1))
    w4t = jnp.transpose(conv4_w.reshape(128, 64, 9), (2, 0, 1))

    args = (x, conv1_w, conv1_b, w2k, conv2_b, w3t, conv3_b,
            w4t, conv4_b, conv5_w, conv5_b,
            linear0_w, linear0_b, linear1_w, linear1_b,
            jnp.asarray(_SW1), jnp.asarray(_SH1), jnp.asarray(_SW2),
            jnp.asarray(_SH2), jnp.asarray(_S3F),
            jnp.asarray(_BMAT), jnp.asarray(_TMASK), jnp.asarray(_ONES8))

    def full(shape):
        nd = len(shape)
        return pl.BlockSpec(shape, lambda i: (0,) * nd)
    in_specs = [pl.BlockSpec((1, 1, 182, 128), lambda i: (0, 0, 0, 0))]
    in_specs += [full(a.shape) for a in args[1:]]
    out = pl.pallas_call(
        _fwd_kernel,
        out_shape=jax.ShapeDtypeStruct((1, 7816), jnp.float32),
        grid=(1,),
        in_specs=in_specs,
        out_specs=full((1, 7816)),
        compiler_params=pltpu.CompilerParams(
            dimension_semantics=("arbitrary",),
            vmem_limit_bytes=_VMEM_LIMIT,
        ),
    )(*args)
    return out
